# baseline (speedup 1.0000x reference)
import functools

import jax
import jax.numpy as jnp
from jax import lax
from jax.experimental import pallas as pl
from jax.experimental.pallas import tpu as pltpu
from jax.experimental.pallas import tpu_sc as plsc

N0, N1, N2, N3 = 100000, 20000, 5000, 1024
E1, E2, E3 = 320000, 80000, 16384
D = 128
NC, NS = 2, 16
CHUNK = 128
SEG = 1024
GROUPS = SEG // 16
CNT_ROWS = 128


def _ceil_to(v, m):
    return (v + m - 1) // m * m


def _make_seg_sum(ndst, e_pad, npass):
    nq = NC * npass
    quarter = ndst // nq
    quarter_out = _ceil_to(quarter, 128)
    garbage = quarter_out
    acc_rows = _ceil_to(quarter_out + 1, 128)
    shift = quarter_out.bit_length()
    e_per_tile = e_pad // NS
    assert e_per_tile % SEG == 0
    n_seg = e_per_tile // SEG
    top = e_per_tile + 2 * CHUNK + 16
    trash = top
    cap = top + 16
    stripe_rows = acc_rows // NS
    assert stripe_rows % 8 == 0
    zn16, zrm = stripe_rows // 16, stripe_rows % 16
    rpt = quarter_out // NS
    assert rpt % 8 == 0
    crpt = CNT_ROWS // NS
    cl_rows = _ceil_to(quarter_out // 128 + 1, 16)
    ctrash = quarter_out // 128
    assert cl_rows <= CNT_ROWS

    mesh = plsc.VectorSubcoreMesh(core_axis_name="c", subcore_axis_name="s",
                                  num_cores=NC, num_subcores=NS)

    @functools.partial(
        pl.kernel,
        out_type=(jax.ShapeDtypeStruct((nq * quarter_out, D), jnp.float32),
                  jax.ShapeDtypeStruct((nq * CNT_ROWS, D), jnp.float32)),
        mesh=mesh,
        compiler_params=pltpu.CompilerParams(needs_layout_passes=False),
        scratch_types=[
            pltpu.VMEM((SEG,), jnp.int32),
            pltpu.VMEM((SEG,), jnp.int32),
            pltpu.VMEM((cap,), jnp.int32),
            pltpu.VMEM((2, CHUNK), jnp.int32),
            pltpu.VMEM((2, CHUNK), jnp.int32),
            pltpu.VMEM((2, CHUNK, D), jnp.float32),
            pltpu.VMEM((npass * cl_rows, D), jnp.float32),
            pltpu.VMEM((cl_rows,), jnp.int32),
            pltpu.VMEM((16, D), jnp.float32),
            pltpu.SemaphoreType.DMA((2,)),
            pltpu.SemaphoreType.DMA((2,)),
            pltpu.SemaphoreType.DMA,
            pltpu.VMEM_SHARED((acc_rows, D), jnp.float32),
            pltpu.VMEM_SHARED((CNT_ROWS, D), jnp.float32),
        ],
    )
    def seg(src_hbm, dst_hbm, x_hbm, out_hbm, cnt_hbm,
            srcraw_v, dstraw_v, spk_v, idx_v, dloc_v, rows_v, cntloc_v,
            rowidx_v, zrow_v, sem_g, sem_s, sem_z,
            acc_sh, cnt_sh):
        c = lax.axis_index("c")
        s = lax.axis_index("s")
        zf16 = jnp.zeros((16,), jnp.float32)
        of16 = jnp.ones((16,), jnp.float32)
        i16 = lax.iota(jnp.int32, 16)

        for r in range(16):
            for j in range(D // 16):
                zrow_v[r, pl.ds(j * 16, 16)] = zf16
        for q in range(cl_rows // 16):
            rowidx_v[pl.ds(q * 16, 16)] = i16 + q * 16

        zbase = s * stripe_rows
        tb = s * e_per_tile

        def fire_zeros():
            for q in range(zn16):
                pltpu.async_copy(zrow_v,
                                 acc_sh.at[pl.ds(zbase + q * 16, 16)], sem_z)
            if zrm:
                pltpu.async_copy(
                    zrow_v.at[pl.ds(0, zrm)],
                    acc_sh.at[pl.ds(zbase + zn16 * 16, zrm)], sem_z)
            pltpu.async_copy(zrow_v.at[pl.ds(0, crpt)],
                             cnt_sh.at[pl.ds(s * crpt, crpt)], sem_z)

        def drain_zeros():
            for q in range(zn16):
                pltpu.make_async_copy(
                    zrow_v, acc_sh.at[pl.ds(zbase + q * 16, 16)],
                    sem_z).wait()
            if zrm:
                pltpu.make_async_copy(
                    zrow_v.at[pl.ds(0, zrm)],
                    acc_sh.at[pl.ds(zbase + zn16 * 16, zrm)], sem_z).wait()
            pltpu.make_async_copy(
                zrow_v.at[pl.ds(0, crpt)],
                cnt_sh.at[pl.ds(s * crpt, crpt)], sem_z).wait()

        fire_zeros()

        def zfill(r, carry):
            for j in range(D // 16):
                cntloc_v[r, pl.ds(j * 16, 16)] = zf16
            return carry
        lax.fori_loop(0, npass * cl_rows, zfill, 0)

        lo0 = c * npass * quarter
        gv16 = jnp.full((16,), garbage, jnp.int32)

        def gbody(g, cs):
            dd = dstraw_v[pl.ds(g * 16, 16)]
            ss = srcraw_v[pl.ds(g * 16, 16)]
            ld0 = dd - lo0
            m0 = (ld0 >= 0) & (ld0 < quarter)
            mi0 = jnp.where(m0, 1, 0)
            cs0 = cs[0]
            pos0 = cs0 + plsc.cumsum(mi0) - mi0
            pos0 = jnp.where(m0, pos0, trash + i16)
            plsc.store_scatter(
                spk_v, [pos0],
                jnp.bitwise_or(lax.shift_left(ss, shift), ld0))
            row0 = jnp.where(m0, lax.shift_right_arithmetic(ld0, 7), ctrash)
            col0 = jnp.where(m0, jnp.bitwise_and(ld0, 127), i16)
            plsc.addupdate_scatter(cntloc_v, [row0, col0], of16)
            cs0 = cs0 + plsc.all_reduce_population_count(m0)
            if npass == 1:
                return (cs0,)
            ld1 = ld0 - quarter
            m1 = (ld1 >= 0) & (ld1 < quarter)
            mi1 = jnp.where(m1, 1, 0)
            cs1 = cs[1]
            pos1 = top - 1 - (cs1 + plsc.cumsum(mi1) - mi1)
            pos1 = jnp.where(m1, pos1, trash + i16)
            plsc.store_scatter(
                spk_v, [pos1],
                jnp.bitwise_or(lax.shift_left(ss, shift), ld1))
            row1 = jnp.where(m1, cl_rows + lax.shift_right_arithmetic(ld1, 7),
                             cl_rows + ctrash)
            col1 = jnp.where(m1, jnp.bitwise_and(ld1, 127), i16)
            plsc.addupdate_scatter(cntloc_v, [row1, col1], of16)
            return (cs0, cs1 + plsc.all_reduce_population_count(m1))

        def sbody(si, cs):
            pltpu.sync_copy(src_hbm.at[pl.ds(tb + si * SEG, SEG)],
                            srcraw_v)
            pltpu.sync_copy(dst_hbm.at[pl.ds(tb + si * SEG, SEG)],
                            dstraw_v)
            return lax.fori_loop(0, GROUPS, gbody, cs, unroll=4)

        z16i = jnp.zeros((16,), jnp.int32)
        csv = lax.fori_loop(0, n_seg, sbody,
                            tuple(z16i for _ in range(npass)))
        cnt_surs = [v[0] for v in csv]
        for j in range(CHUNK // 16):
            spk_v[pl.ds(cnt_surs[0] + j * 16, 16)] = gv16
        if npass == 2:
            for j in range(CHUNK // 16):
                spk_v[pl.ds(top - cnt_surs[1] - CHUNK + j * 16, 16)] = gv16

        drain_zeros()

        for p in range(npass):
            cnt_sur = cnt_surs[p]
            n_ch = lax.shift_right_logical(cnt_sur + (CHUNK - 1), 7)
            qid = c * npass + p

            plsc.subcore_barrier()
            pltpu.sync_copy(cntloc_v.at[pl.ds(p * cl_rows, cl_rows)],
                            cnt_sh.at[rowidx_v], add=True)

            def stage(k, b):
                for j in range(CHUNK // 16):
                    if p == 0:
                        pv = spk_v[pl.ds(k * CHUNK + j * 16, 16)]
                    else:
                        pv = spk_v[pl.ds(top - (k + 1) * CHUNK + j * 16, 16)]
                    idx_v[b, pl.ds(j * 16, 16)] = \
                        lax.shift_right_logical(pv, shift)
                    dloc_v[b, pl.ds(j * 16, 16)] = \
                        jnp.bitwise_and(pv, (1 << shift) - 1)

            def fire_gather(b):
                pltpu.async_copy(x_hbm.at[idx_v.at[b]], rows_v.at[b],
                                 sem_g.at[b])

            def wait_gather(b):
                pltpu.make_async_copy(x_hbm.at[idx_v.at[b]], rows_v.at[b],
                                      sem_g.at[b]).wait()

            def fire_scatter(b):
                pltpu.async_copy(rows_v.at[b], acc_sh.at[dloc_v.at[b]],
                                 sem_s.at[b], add=True)

            def drain_scatter(b):
                pltpu.make_async_copy(rows_v.at[b], acc_sh.at[dloc_v.at[b]],
                                      sem_s.at[b]).wait()

            stage(jnp.int32(0), jnp.int32(0))
            fire_gather(jnp.int32(0))

            def cbody(i, carry):
                b = jnp.bitwise_and(i, 1)
                nb = 1 - b

                @pl.when(i + 1 < n_ch)
                def _():
                    @pl.when(i >= 1)
                    def _():
                        drain_scatter(nb)
                    stage(i + 1, nb)
                    fire_gather(nb)

                wait_gather(b)
                fire_scatter(b)
                return carry

            lax.fori_loop(0, n_ch, cbody, 0)
            last = jnp.bitwise_and(n_ch - 1, 1)

            @pl.when(n_ch >= 2)
            def _():
                drain_scatter(1 - last)
            drain_scatter(last)
            plsc.subcore_barrier()

            pltpu.sync_copy(acc_sh.at[pl.ds(s * rpt, rpt)],
                            out_hbm.at[pl.ds(qid * quarter_out + s * rpt,
                                             rpt)])
            pltpu.sync_copy(cnt_sh.at[pl.ds(s * crpt, crpt)],
                            cnt_hbm.at[pl.ds(qid * CNT_ROWS + s * crpt,
                                             crpt)])
            if p + 1 < npass:
                plsc.subcore_barrier()
                fire_zeros()
                drain_zeros()

    return seg, quarter, quarter_out, nq


_EPAD = {N1: _ceil_to(E1, NS * SEG),
         N2: _ceil_to(E2, NS * SEG),
         N3: _ceil_to(E3, NS * SEG)}
_SEGS = {N1: _make_seg_sum(N1, _EPAD[N1], 2),
         N2: _make_seg_sum(N2, _EPAD[N2], 1),
         N3: _make_seg_sum(N3, _EPAD[N3], 1)}


def _tc_layer(summed, cnt2, x_tgt, wlT, bl2, wrT, a2, n):
    BR = 512
    grid = (n + BR - 1) // BR

    def body(s_ref, c_ref, xt_ref, wl_ref, bl_ref, wr_ref, a_ref, o_ref):
        ct = c_ref[...]
        mean = s_ref[...] / jnp.maximum(ct, 1.0)
        y = jnp.dot(mean, wl_ref[...], preferred_element_type=jnp.float32,
                    precision=lax.Precision.HIGHEST)
        y = y + jnp.dot(xt_ref[...], wr_ref[...],
                        preferred_element_type=jnp.float32,
                        precision=lax.Precision.HIGHEST)
        y = y + bl_ref[...]
        o_ref[...] = jnp.where(y > 0.0, y, a_ref[...] * y)

    return pl.pallas_call(
        body,
        grid=(grid,),
        in_specs=[
            pl.BlockSpec((BR, D), lambda i: (i, 0)),
            pl.BlockSpec((BR, 1), lambda i: (i, 0)),
            pl.BlockSpec((BR, D), lambda i: (i, 0)),
            pl.BlockSpec((D, D), lambda i: (0, 0)),
            pl.BlockSpec((1, D), lambda i: (0, 0)),
            pl.BlockSpec((D, D), lambda i: (0, 0)),
            pl.BlockSpec((1, D), lambda i: (0, 0)),
        ],
        out_specs=pl.BlockSpec((BR, D), lambda i: (i, 0)),
        out_shape=jax.ShapeDtypeStruct((n, D), jnp.float32),
    )(summed, cnt2, x_tgt, wlT, bl2, wrT, a2)


def _layer(x_src, x_tgt, edge_index, ndst, Wl, bl, Wr, a):
    seg, quarter, quarter_out, nq = _SEGS[ndst]
    e_pad = _EPAD[ndst]
    src = edge_index[0]
    dst = edge_index[1]
    padn = e_pad - src.shape[0]
    if padn:
        src = jnp.pad(src, (0, padn))
        dst = jnp.pad(dst, (0, padn), constant_values=-1)
    flat, cnt2 = seg(src, dst, x_src)
    if nq == 1 and quarter == quarter_out:
        summed = flat
    else:
        summed = jnp.concatenate(
            [flat[q * quarter_out:q * quarter_out + quarter]
             for q in range(nq)], axis=0)
    cntf = cnt2.reshape(-1)
    cl = CNT_ROWS * D
    cnt = jnp.concatenate(
        [cntf[q * cl:q * cl + quarter] for q in range(nq)])
    return _tc_layer(summed, cnt[:, None], x_tgt, Wl.T, bl[None, :], Wr.T,
                     a[None, :], ndst)


def kernel(x, edge_index1, edge_index2, edge_index3,
           Wl1, bl1, Wr1, a1, Wl2, bl2, Wr2, a2, Wl3, bl3, Wr3, a3):
    h1 = _layer(x, x[:N1], edge_index1, N1, Wl1, bl1, Wr1, a1)
    h2 = _layer(h1, h1[:N2], edge_index2, N2, Wl2, bl2, Wr2, a2)
    h3 = _layer(h2, h2[:N3], edge_index3, N3, Wl3, bl3, Wr3, a3)
    return h3


# speedup vs baseline: 1.0005x; 1.0005x over previous
import functools

import jax
import jax.numpy as jnp
from jax import lax
from jax.experimental import pallas as pl
from jax.experimental.pallas import tpu as pltpu
from jax.experimental.pallas import tpu_sc as plsc

N0, N1, N2, N3 = 100000, 20000, 5000, 1024
E1, E2, E3 = 320000, 80000, 16384
D = 128
NC, NS = 2, 16
CHUNK = 128
SEG = 1024
GROUPS = SEG // 16
CNT_ROWS = 128


def _ceil_to(v, m):
    return (v + m - 1) // m * m


def _make_seg_sum(ndst, e_pad, npass):
    nq = NC * npass
    quarter = ndst // nq
    quarter_out = _ceil_to(quarter, 128)
    garbage = quarter_out
    acc_rows = _ceil_to(quarter_out + 1, 128)
    shift = quarter_out.bit_length()
    e_per_tile = e_pad // NS
    assert e_per_tile % SEG == 0
    n_seg = e_per_tile // SEG
    top = e_per_tile + 2 * CHUNK + 16
    trash = top
    cap = top + 16
    stripe_rows = acc_rows // NS
    assert stripe_rows % 8 == 0
    zn16, zrm = stripe_rows // 16, stripe_rows % 16
    rpt = quarter_out // NS
    assert rpt % 8 == 0
    crpt = CNT_ROWS // NS
    cl_rows = _ceil_to(quarter_out // 128 + 1, 16)
    ctrash = quarter_out // 128
    assert cl_rows <= CNT_ROWS

    mesh = plsc.VectorSubcoreMesh(core_axis_name="c", subcore_axis_name="s",
                                  num_cores=NC, num_subcores=NS)

    @functools.partial(
        pl.kernel,
        out_type=(jax.ShapeDtypeStruct((nq * quarter_out, D), jnp.float32),
                  jax.ShapeDtypeStruct((nq * CNT_ROWS, D), jnp.float32)),
        mesh=mesh,
        compiler_params=pltpu.CompilerParams(needs_layout_passes=False),
        scratch_types=[
            pltpu.VMEM((SEG,), jnp.int32),
            pltpu.VMEM((SEG,), jnp.int32),
            pltpu.VMEM((cap,), jnp.int32),
            pltpu.VMEM((2, CHUNK), jnp.int32),
            pltpu.VMEM((2, CHUNK), jnp.int32),
            pltpu.VMEM((2, CHUNK, D), jnp.float32),
            pltpu.VMEM((npass * cl_rows, D), jnp.float32),
            pltpu.VMEM((cl_rows,), jnp.int32),
            pltpu.VMEM((16, D), jnp.float32),
            pltpu.SemaphoreType.DMA((2,)),
            pltpu.SemaphoreType.DMA((2,)),
            pltpu.SemaphoreType.DMA,
            pltpu.VMEM_SHARED((acc_rows, D), jnp.float32),
            pltpu.VMEM_SHARED((CNT_ROWS, D), jnp.float32),
        ],
    )
    def seg(src_hbm, dst_hbm, x_hbm, out_hbm, cnt_hbm,
            srcraw_v, dstraw_v, spk_v, idx_v, dloc_v, rows_v, cntloc_v,
            rowidx_v, zrow_v, sem_g, sem_s, sem_z,
            acc_sh, cnt_sh):
        c = lax.axis_index("c")
        s = lax.axis_index("s")
        zf16 = jnp.zeros((16,), jnp.float32)
        of16 = jnp.ones((16,), jnp.float32)
        i16 = lax.iota(jnp.int32, 16)

        for r in range(16):
            for j in range(D // 16):
                zrow_v[r, pl.ds(j * 16, 16)] = zf16
        for q in range(cl_rows // 16):
            rowidx_v[pl.ds(q * 16, 16)] = i16 + q * 16

        zbase = s * stripe_rows
        tb = s * e_per_tile

        def fire_zeros():
            for q in range(zn16):
                pltpu.async_copy(zrow_v,
                                 acc_sh.at[pl.ds(zbase + q * 16, 16)], sem_z)
            if zrm:
                pltpu.async_copy(
                    zrow_v.at[pl.ds(0, zrm)],
                    acc_sh.at[pl.ds(zbase + zn16 * 16, zrm)], sem_z)
            pltpu.async_copy(zrow_v.at[pl.ds(0, crpt)],
                             cnt_sh.at[pl.ds(s * crpt, crpt)], sem_z)

        def drain_zeros():
            for q in range(zn16):
                pltpu.make_async_copy(
                    zrow_v, acc_sh.at[pl.ds(zbase + q * 16, 16)],
                    sem_z).wait()
            if zrm:
                pltpu.make_async_copy(
                    zrow_v.at[pl.ds(0, zrm)],
                    acc_sh.at[pl.ds(zbase + zn16 * 16, zrm)], sem_z).wait()
            pltpu.make_async_copy(
                zrow_v.at[pl.ds(0, crpt)],
                cnt_sh.at[pl.ds(s * crpt, crpt)], sem_z).wait()

        fire_zeros()

        def zfill(r, carry):
            for j in range(D // 16):
                cntloc_v[r, pl.ds(j * 16, 16)] = zf16
            return carry
        lax.fori_loop(0, npass * cl_rows, zfill, 0)

        lo0 = c * npass * quarter
        gv16 = jnp.full((16,), garbage, jnp.int32)

        def gbody(g, cs):
            dd = dstraw_v[pl.ds(g * 16, 16)]
            ss = srcraw_v[pl.ds(g * 16, 16)]
            ld0 = dd - lo0
            m0 = (ld0 >= 0) & (ld0 < quarter)
            mi0 = jnp.where(m0, 1, 0)
            cs0 = cs[0]
            pos0 = cs0 + plsc.cumsum(mi0) - mi0
            pos0 = jnp.where(m0, pos0, trash + i16)
            plsc.store_scatter(
                spk_v, [pos0],
                jnp.bitwise_or(lax.shift_left(ss, shift), ld0))
            row0 = jnp.where(m0, lax.shift_right_arithmetic(ld0, 7), ctrash)
            col0 = jnp.where(m0, jnp.bitwise_and(ld0, 127), i16)
            plsc.addupdate_scatter(cntloc_v, [row0, col0], of16)
            cs0 = cs0 + plsc.all_reduce_population_count(m0)
            if npass == 1:
                return (cs0,)
            ld1 = ld0 - quarter
            m1 = (ld1 >= 0) & (ld1 < quarter)
            mi1 = jnp.where(m1, 1, 0)
            cs1 = cs[1]
            pos1 = top - 1 - (cs1 + plsc.cumsum(mi1) - mi1)
            pos1 = jnp.where(m1, pos1, trash + i16)
            plsc.store_scatter(
                spk_v, [pos1],
                jnp.bitwise_or(lax.shift_left(ss, shift), ld1))
            row1 = jnp.where(m1, cl_rows + lax.shift_right_arithmetic(ld1, 7),
                             cl_rows + ctrash)
            col1 = jnp.where(m1, jnp.bitwise_and(ld1, 127), i16)
            plsc.addupdate_scatter(cntloc_v, [row1, col1], of16)
            return (cs0, cs1 + plsc.all_reduce_population_count(m1))

        def sbody(si, cs):
            pltpu.sync_copy(src_hbm.at[pl.ds(tb + si * SEG, SEG)],
                            srcraw_v)
            pltpu.sync_copy(dst_hbm.at[pl.ds(tb + si * SEG, SEG)],
                            dstraw_v)
            return lax.fori_loop(0, GROUPS, gbody, cs, unroll=4)

        z16i = jnp.zeros((16,), jnp.int32)
        csv = lax.fori_loop(0, n_seg, sbody,
                            tuple(z16i for _ in range(npass)))
        cnt_surs = [v[0] for v in csv]
        for j in range(CHUNK // 16):
            spk_v[pl.ds(cnt_surs[0] + j * 16, 16)] = gv16
        if npass == 2:
            for j in range(CHUNK // 16):
                spk_v[pl.ds(top - cnt_surs[1] - CHUNK + j * 16, 16)] = gv16

        drain_zeros()

        for p in range(npass):
            cnt_sur = cnt_surs[p]
            n_ch = jnp.maximum(
                lax.shift_right_logical(cnt_sur + (CHUNK - 1), 7), 1)
            qid = c * npass + p

            plsc.subcore_barrier()
            pltpu.sync_copy(cntloc_v.at[pl.ds(p * cl_rows, cl_rows)],
                            cnt_sh.at[rowidx_v], add=True)

            def stage(k, b):
                for j in range(CHUNK // 16):
                    if p == 0:
                        pv = spk_v[pl.ds(k * CHUNK + j * 16, 16)]
                    else:
                        pv = spk_v[pl.ds(top - (k + 1) * CHUNK + j * 16, 16)]
                    idx_v[b, pl.ds(j * 16, 16)] = \
                        lax.shift_right_logical(pv, shift)
                    dloc_v[b, pl.ds(j * 16, 16)] = \
                        jnp.bitwise_and(pv, (1 << shift) - 1)

            def fire_gather(b):
                pltpu.async_copy(x_hbm.at[idx_v.at[b]], rows_v.at[b],
                                 sem_g.at[b])

            def wait_gather(b):
                pltpu.make_async_copy(x_hbm.at[idx_v.at[b]], rows_v.at[b],
                                      sem_g.at[b]).wait()

            def fire_scatter(b):
                pltpu.async_copy(rows_v.at[b], acc_sh.at[dloc_v.at[b]],
                                 sem_s.at[b], add=True)

            def drain_scatter(b):
                pltpu.make_async_copy(rows_v.at[b], acc_sh.at[dloc_v.at[b]],
                                      sem_s.at[b]).wait()

            stage(jnp.int32(0), jnp.int32(0))
            fire_gather(jnp.int32(0))

            def cbody(i, carry):
                b = jnp.bitwise_and(i, 1)
                nb = 1 - b

                @pl.when(i + 1 < n_ch)
                def _():
                    @pl.when(i >= 1)
                    def _():
                        drain_scatter(nb)
                    stage(i + 1, nb)
                    fire_gather(nb)

                wait_gather(b)
                fire_scatter(b)
                return carry

            lax.fori_loop(0, n_ch, cbody, 0)
            last = jnp.bitwise_and(n_ch - 1, 1)

            @pl.when(n_ch >= 2)
            def _():
                drain_scatter(1 - last)
            drain_scatter(last)
            plsc.subcore_barrier()

            pltpu.sync_copy(acc_sh.at[pl.ds(s * rpt, rpt)],
                            out_hbm.at[pl.ds(qid * quarter_out + s * rpt,
                                             rpt)])
            pltpu.sync_copy(cnt_sh.at[pl.ds(s * crpt, crpt)],
                            cnt_hbm.at[pl.ds(qid * CNT_ROWS + s * crpt,
                                             crpt)])
            if p + 1 < npass:
                plsc.subcore_barrier()
                fire_zeros()
                drain_zeros()

    return seg, quarter, quarter_out, nq


_EPAD = {N1: _ceil_to(E1, NS * SEG),
         N2: _ceil_to(E2, NS * SEG),
         N3: _ceil_to(E3, NS * SEG)}
_SEGS = {N1: _make_seg_sum(N1, _EPAD[N1], 2),
         N2: _make_seg_sum(N2, _EPAD[N2], 1),
         N3: _make_seg_sum(N3, _EPAD[N3], 1)}


def _tc_layer(summed, cnt2, x_tgt, wlT, bl2, wrT, a2, n):
    BR = 512
    grid = (n + BR - 1) // BR

    def body(s_ref, c_ref, xt_ref, wl_ref, bl_ref, wr_ref, a_ref, o_ref):
        ct = c_ref[...]
        mean = s_ref[...] / jnp.maximum(ct, 1.0)
        y = jnp.dot(mean, wl_ref[...], preferred_element_type=jnp.float32,
                    precision=lax.Precision.HIGHEST)
        y = y + jnp.dot(xt_ref[...], wr_ref[...],
                        preferred_element_type=jnp.float32,
                        precision=lax.Precision.HIGHEST)
        y = y + bl_ref[...]
        o_ref[...] = jnp.where(y > 0.0, y, a_ref[...] * y)

    return pl.pallas_call(
        body,
        grid=(grid,),
        in_specs=[
            pl.BlockSpec((BR, D), lambda i: (i, 0)),
            pl.BlockSpec((BR, 1), lambda i: (i, 0)),
            pl.BlockSpec((BR, D), lambda i: (i, 0)),
            pl.BlockSpec((D, D), lambda i: (0, 0)),
            pl.BlockSpec((1, D), lambda i: (0, 0)),
            pl.BlockSpec((D, D), lambda i: (0, 0)),
            pl.BlockSpec((1, D), lambda i: (0, 0)),
        ],
        out_specs=pl.BlockSpec((BR, D), lambda i: (i, 0)),
        out_shape=jax.ShapeDtypeStruct((n, D), jnp.float32),
    )(summed, cnt2, x_tgt, wlT, bl2, wrT, a2)


def _layer(x_src, x_tgt, edge_index, ndst, Wl, bl, Wr, a):
    seg, quarter, quarter_out, nq = _SEGS[ndst]
    e_pad = _EPAD[ndst]
    src = edge_index[0]
    dst = edge_index[1]
    padn = e_pad - src.shape[0]
    if padn:
        src = jnp.pad(src, (0, padn))
        dst = jnp.pad(dst, (0, padn), constant_values=-1)
    flat, cnt2 = seg(src, dst, x_src)
    if nq == 1 and quarter == quarter_out:
        summed = flat
    else:
        summed = jnp.concatenate(
            [flat[q * quarter_out:q * quarter_out + quarter]
             for q in range(nq)], axis=0)
    cntf = cnt2.reshape(-1)
    cl = CNT_ROWS * D
    cnt = jnp.concatenate(
        [cntf[q * cl:q * cl + quarter] for q in range(nq)])
    return _tc_layer(summed, cnt[:, None], x_tgt, Wl.T, bl[None, :], Wr.T,
                     a[None, :], ndst)


def kernel(x, edge_index1, edge_index2, edge_index3,
           Wl1, bl1, Wr1, a1, Wl2, bl2, Wr2, a2, Wl3, bl3, Wr3, a3):
    h1 = _layer(x, x[:N1], edge_index1, N1, Wl1, bl1, Wr1, a1)
    h2 = _layer(h1, h1[:N2], edge_index2, N2, Wl2, bl2, Wr2, a2)
    h3 = _layer(h2, h2[:N3], edge_index3, N3, Wl3, bl3, Wr3, a3)
    return h3


# speedup vs baseline: 1.1073x; 1.1068x over previous
import functools

import jax
import jax.numpy as jnp
from jax import lax
from jax.experimental import pallas as pl
from jax.experimental.pallas import tpu as pltpu
from jax.experimental.pallas import tpu_sc as plsc

N0, N1, N2, N3 = 100000, 20000, 5000, 1024
E1, E2, E3 = 320000, 80000, 16384
D = 128
NC, NS = 2, 16
CHUNK = 128
SEG = 1024
GROUPS = SEG // 16
CNT_ROWS = 128


def _ceil_to(v, m):
    return (v + m - 1) // m * m


def _make_seg_sum(ndst, e_pad, npass):
    nq = NC * npass
    quarter = ndst // nq
    quarter_out = _ceil_to(quarter, 128)
    garbage = quarter_out
    acc_rows = _ceil_to(quarter_out + 1, 128)
    shift = quarter_out.bit_length()
    e_per_tile = e_pad // NS
    assert e_per_tile % SEG == 0
    n_seg = e_per_tile // SEG
    top = e_per_tile + 2 * CHUNK + 16
    trash = top
    cap = top + 16
    stripe_rows = acc_rows // NS
    assert stripe_rows % 8 == 0
    zn16, zrm = stripe_rows // 16, stripe_rows % 16
    rpt = quarter_out // NS
    assert rpt % 8 == 0
    crpt = CNT_ROWS // NS
    cl_rows = _ceil_to(quarter_out // 128 + 1, 16)
    ctrash = quarter_out // 128
    assert cl_rows <= CNT_ROWS

    mesh = plsc.VectorSubcoreMesh(core_axis_name="c", subcore_axis_name="s",
                                  num_cores=NC, num_subcores=NS)

    @functools.partial(
        pl.kernel,
        out_type=(jax.ShapeDtypeStruct((nq * quarter_out, D), jnp.float32),
                  jax.ShapeDtypeStruct((nq * CNT_ROWS, D), jnp.float32)),
        mesh=mesh,
        compiler_params=pltpu.CompilerParams(needs_layout_passes=False),
        scratch_types=[
            pltpu.VMEM((SEG,), jnp.int32),
            pltpu.VMEM((SEG,), jnp.int32),
            pltpu.VMEM((cap,), jnp.int32),
            pltpu.VMEM((2, CHUNK), jnp.int32),
            pltpu.VMEM((2, CHUNK), jnp.int32),
            pltpu.VMEM((2, CHUNK, D), jnp.float32),
            pltpu.VMEM((npass * cl_rows, D), jnp.float32),
            pltpu.VMEM((cl_rows,), jnp.int32),
            pltpu.VMEM((16, D), jnp.float32),
            pltpu.SemaphoreType.DMA((2,)),
            pltpu.SemaphoreType.DMA((2,)),
            pltpu.SemaphoreType.DMA,
            pltpu.VMEM_SHARED((acc_rows, D), jnp.float32),
            pltpu.VMEM_SHARED((CNT_ROWS, D), jnp.float32),
        ],
    )
    def seg(src_hbm, dst_hbm, x_hbm, out_hbm, cnt_hbm,
            srcraw_v, dstraw_v, spk_v, idx_v, dloc_v, rows_v, cntloc_v,
            rowidx_v, zrow_v, sem_g, sem_s, sem_z,
            acc_sh, cnt_sh):
        c = lax.axis_index("c")
        s = lax.axis_index("s")
        zf16 = jnp.zeros((16,), jnp.float32)
        of16 = jnp.ones((16,), jnp.float32)
        i16 = lax.iota(jnp.int32, 16)

        for r in range(16):
            for j in range(D // 16):
                zrow_v[r, pl.ds(j * 16, 16)] = zf16
        for q in range(cl_rows // 16):
            rowidx_v[pl.ds(q * 16, 16)] = i16 + q * 16

        zbase = s * stripe_rows
        tb = s * e_per_tile

        def fire_zeros():
            for q in range(zn16):
                pltpu.async_copy(zrow_v,
                                 acc_sh.at[pl.ds(zbase + q * 16, 16)], sem_z)
            if zrm:
                pltpu.async_copy(
                    zrow_v.at[pl.ds(0, zrm)],
                    acc_sh.at[pl.ds(zbase + zn16 * 16, zrm)], sem_z)
            pltpu.async_copy(zrow_v.at[pl.ds(0, crpt)],
                             cnt_sh.at[pl.ds(s * crpt, crpt)], sem_z)

        def drain_zeros():
            for q in range(zn16):
                pltpu.make_async_copy(
                    zrow_v, acc_sh.at[pl.ds(zbase + q * 16, 16)],
                    sem_z).wait()
            if zrm:
                pltpu.make_async_copy(
                    zrow_v.at[pl.ds(0, zrm)],
                    acc_sh.at[pl.ds(zbase + zn16 * 16, zrm)], sem_z).wait()
            pltpu.make_async_copy(
                zrow_v.at[pl.ds(0, crpt)],
                cnt_sh.at[pl.ds(s * crpt, crpt)], sem_z).wait()

        fire_zeros()

        def zfill(r, carry):
            for j in range(D // 16):
                cntloc_v[r, pl.ds(j * 16, 16)] = zf16
            return carry
        lax.fori_loop(0, npass * cl_rows, zfill, 0)

        lo0 = c * npass * quarter
        gv16 = jnp.full((16,), garbage, jnp.int32)

        def gbody(g, cs):
            dd = dstraw_v[pl.ds(g * 16, 16)]
            ss = srcraw_v[pl.ds(g * 16, 16)]
            ld0 = dd - lo0
            m0 = (ld0 >= 0) & (ld0 < quarter)
            mi0 = jnp.where(m0, 1, 0)
            cs0 = cs[0]
            pos0 = cs0 + plsc.cumsum(mi0) - mi0
            pos0 = jnp.where(m0, pos0, trash + i16)
            plsc.store_scatter(
                spk_v, [pos0],
                jnp.bitwise_or(lax.shift_left(ss, shift), ld0))
            row0 = jnp.where(m0, lax.shift_right_arithmetic(ld0, 7), ctrash)
            col0 = jnp.where(m0, jnp.bitwise_and(ld0, 127), i16)
            plsc.addupdate_scatter(cntloc_v, [row0, col0], of16)
            cs0 = cs0 + plsc.all_reduce_population_count(m0)
            if npass == 1:
                return (cs0,)
            ld1 = ld0 - quarter
            m1 = (ld1 >= 0) & (ld1 < quarter)
            mi1 = jnp.where(m1, 1, 0)
            cs1 = cs[1]
            pos1 = top - 1 - (cs1 + plsc.cumsum(mi1) - mi1)
            pos1 = jnp.where(m1, pos1, trash + i16)
            plsc.store_scatter(
                spk_v, [pos1],
                jnp.bitwise_or(lax.shift_left(ss, shift), ld1))
            row1 = jnp.where(m1, cl_rows + lax.shift_right_arithmetic(ld1, 7),
                             cl_rows + ctrash)
            col1 = jnp.where(m1, jnp.bitwise_and(ld1, 127), i16)
            plsc.addupdate_scatter(cntloc_v, [row1, col1], of16)
            return (cs0, cs1 + plsc.all_reduce_population_count(m1))

        def sbody(si, cs):
            pltpu.sync_copy(src_hbm.at[pl.ds(tb + si * SEG, SEG)],
                            srcraw_v)
            pltpu.sync_copy(dst_hbm.at[pl.ds(tb + si * SEG, SEG)],
                            dstraw_v)
            return lax.fori_loop(0, GROUPS, gbody, cs, unroll=4)

        z16i = jnp.zeros((16,), jnp.int32)
        csv = lax.fori_loop(0, n_seg, sbody,
                            tuple(z16i for _ in range(npass)))
        cnt_surs = [v[0] for v in csv]
        for j in range(CHUNK // 16):
            spk_v[pl.ds(cnt_surs[0] + j * 16, 16)] = gv16
        if npass == 2:
            for j in range(CHUNK // 16):
                spk_v[pl.ds(top - cnt_surs[1] - CHUNK + j * 16, 16)] = gv16

        drain_zeros()

        for p in range(npass):
            cnt_sur = cnt_surs[p]
            n_ch = jnp.maximum(
                lax.shift_right_logical(cnt_sur + (CHUNK - 1), 7), 1)
            qid = c * npass + p

            plsc.subcore_barrier()
            pltpu.sync_copy(cntloc_v.at[pl.ds(p * cl_rows, cl_rows)],
                            cnt_sh.at[rowidx_v], add=True)

            def stage(k, b):
                for j in range(CHUNK // 16):
                    if p == 0:
                        pv = spk_v[pl.ds(k * CHUNK + j * 16, 16)]
                    else:
                        pv = spk_v[pl.ds(top - (k + 1) * CHUNK + j * 16, 16)]
                    idx_v[b, pl.ds(j * 16, 16)] = \
                        lax.shift_right_logical(pv, shift)
                    dloc_v[b, pl.ds(j * 16, 16)] = \
                        jnp.bitwise_and(pv, (1 << shift) - 1)

            def fire_gather(b):
                pltpu.async_copy(x_hbm.at[idx_v.at[b]], rows_v.at[b],
                                 sem_g.at[b])

            def wait_gather(b):
                pltpu.make_async_copy(x_hbm.at[idx_v.at[b]], rows_v.at[b],
                                      sem_g.at[b]).wait()

            def fire_scatter(b):
                pltpu.async_copy(rows_v.at[b], acc_sh.at[dloc_v.at[b]],
                                 sem_s.at[b], add=True)

            def drain_scatter(b):
                pltpu.make_async_copy(rows_v.at[b], acc_sh.at[dloc_v.at[b]],
                                      sem_s.at[b]).wait()

            stage(jnp.int32(0), jnp.int32(0))
            fire_gather(jnp.int32(0))

            def cbody(i, carry):
                b = jnp.bitwise_and(i, 1)
                nb = 1 - b

                @pl.when(i + 1 < n_ch)
                def _():
                    @pl.when(i >= 1)
                    def _():
                        drain_scatter(nb)
                    stage(i + 1, nb)
                    fire_gather(nb)

                wait_gather(b)
                fire_scatter(b)
                return carry

            lax.fori_loop(0, n_ch, cbody, 0)
            last = jnp.bitwise_and(n_ch - 1, 1)

            @pl.when(n_ch >= 2)
            def _():
                drain_scatter(1 - last)
            drain_scatter(last)
            plsc.subcore_barrier()

            pltpu.sync_copy(acc_sh.at[pl.ds(s * rpt, rpt)],
                            out_hbm.at[pl.ds(qid * quarter_out + s * rpt,
                                             rpt)])
            pltpu.sync_copy(cnt_sh.at[pl.ds(s * crpt, crpt)],
                            cnt_hbm.at[pl.ds(qid * CNT_ROWS + s * crpt,
                                             crpt)])
            if p + 1 < npass:
                plsc.subcore_barrier()
                fire_zeros()
                drain_zeros()

    return seg, quarter, quarter_out, nq


def _make_seg_simple(ndst, e_pad):
    half = ndst // NC
    half_out = _ceil_to(half, 128)
    garbage = half_out
    acc_rows = _ceil_to(half_out + 1, 128)
    cnt_len = _ceil_to(half_out + 1, 256)
    chunks_per_tile = e_pad // (NS * CHUNK)
    assert e_pad == chunks_per_tile * NS * CHUNK
    stripe_rows = acc_rows // NS
    assert stripe_rows % 8 == 0
    cnt_stripe = cnt_len // NS
    assert cnt_stripe % 16 == 0
    rpt = half_out // NS
    assert rpt % 8 == 0

    mesh = plsc.VectorSubcoreMesh(core_axis_name="c", subcore_axis_name="s",
                                  num_cores=NC, num_subcores=NS)

    @functools.partial(
        pl.kernel,
        out_type=(jax.ShapeDtypeStruct((NC * half_out, D), jnp.float32),
                  jax.ShapeDtypeStruct((NC * half_out,), jnp.float32)),
        mesh=mesh,
        scratch_types=[
            pltpu.VMEM((CHUNK,), jnp.int32),
            pltpu.VMEM((CHUNK,), jnp.int32),
            pltpu.VMEM((CHUNK, D), jnp.float32),
            pltpu.VMEM((CHUNK,), jnp.float32),
            pltpu.VMEM((8, D), jnp.float32),
            pltpu.VMEM((cnt_stripe,), jnp.float32),
            pltpu.VMEM((rpt,), jnp.float32),
            pltpu.SemaphoreType.DMA,
            pltpu.VMEM_SHARED((acc_rows, D), jnp.float32),
            pltpu.VMEM_SHARED((cnt_len,), jnp.float32),
        ],
    )
    def seg(src_hbm, dst_hbm, x_hbm, out_hbm, cnt_hbm,
            idx_v, dloc_v, rows_v, ones_v, zrows_v, zcnt_v, cbuf_v, sem,
            acc_sh, cnt_sh):
        c = lax.axis_index("c")
        s = lax.axis_index("s")
        z16 = jnp.zeros((16,), jnp.float32)
        o16 = jnp.ones((16,), jnp.float32)
        for r in range(8):
            for j in range(D // 16):
                zrows_v[r, pl.ds(j * 16, 16)] = z16
        for j in range(CHUNK // 16):
            ones_v[pl.ds(j * 16, 16)] = o16
        for j in range(cnt_stripe // 16):
            zcnt_v[pl.ds(j * 16, 16)] = z16

        def zbody(i, carry):
            pltpu.sync_copy(zrows_v,
                            acc_sh.at[pl.ds(s * stripe_rows + i * 8, 8)])
            return carry
        lax.fori_loop(0, stripe_rows // 8, zbody, 0)
        pltpu.sync_copy(zcnt_v, cnt_sh.at[pl.ds(s * cnt_stripe, cnt_stripe)])
        plsc.subcore_barrier()

        lo = c * half
        hi = lo + half

        def chunk_body(i, carry):
            base = (s * chunks_per_tile + i) * CHUNK
            pltpu.sync_copy(src_hbm.at[pl.ds(base, CHUNK)], idx_v)
            pltpu.sync_copy(dst_hbm.at[pl.ds(base, CHUNK)], dloc_v)
            for j in range(CHUNK // 16):
                dd = dloc_v[pl.ds(j * 16, 16)]
                oob = (dd < lo) | (dd >= hi)
                dloc_v[pl.ds(j * 16, 16)] = jnp.where(oob, garbage, dd - lo)
            pltpu.async_copy(x_hbm.at[idx_v], rows_v, sem).wait()
            pltpu.sync_copy(rows_v, acc_sh.at[dloc_v], add=True)
            pltpu.sync_copy(ones_v, cnt_sh.at[dloc_v], add=True)
            return carry
        lax.fori_loop(0, chunks_per_tile, chunk_body, 0)
        plsc.subcore_barrier()

        pltpu.sync_copy(acc_sh.at[pl.ds(s * rpt, rpt)],
                        out_hbm.at[pl.ds(c * half_out + s * rpt, rpt)])
        pltpu.sync_copy(cnt_sh.at[pl.ds(s * rpt, rpt)], cbuf_v)
        pltpu.sync_copy(cbuf_v,
                        cnt_hbm.at[pl.ds(c * half_out + s * rpt, rpt)])

    return seg, half, half_out


_EPAD = {N1: _ceil_to(E1, NS * SEG),
         N2: _ceil_to(E2, NS * SEG),
         N3: _ceil_to(E3, NS * SEG)}
_SEGS = {N1: _make_seg_sum(N1, _EPAD[N1], 2),
         N2: _make_seg_sum(N2, _EPAD[N2], 1)}
_SEG3S = _make_seg_simple(N3, E3)


def _tc_layer(summed, cnt2, x_tgt, wlT, bl2, wrT, a2, n):
    BR = 512
    grid = (n + BR - 1) // BR

    def body(s_ref, c_ref, xt_ref, wl_ref, bl_ref, wr_ref, a_ref, o_ref):
        ct = c_ref[...]
        mean = s_ref[...] / jnp.maximum(ct, 1.0)
        y = jnp.dot(mean, wl_ref[...], preferred_element_type=jnp.float32,
                    precision=lax.Precision.HIGHEST)
        y = y + jnp.dot(xt_ref[...], wr_ref[...],
                        preferred_element_type=jnp.float32,
                        precision=lax.Precision.HIGHEST)
        y = y + bl_ref[...]
        o_ref[...] = jnp.where(y > 0.0, y, a_ref[...] * y)

    return pl.pallas_call(
        body,
        grid=(grid,),
        in_specs=[
            pl.BlockSpec((BR, D), lambda i: (i, 0)),
            pl.BlockSpec((BR, 1), lambda i: (i, 0)),
            pl.BlockSpec((BR, D), lambda i: (i, 0)),
            pl.BlockSpec((D, D), lambda i: (0, 0)),
            pl.BlockSpec((1, D), lambda i: (0, 0)),
            pl.BlockSpec((D, D), lambda i: (0, 0)),
            pl.BlockSpec((1, D), lambda i: (0, 0)),
        ],
        out_specs=pl.BlockSpec((BR, D), lambda i: (i, 0)),
        out_shape=jax.ShapeDtypeStruct((n, D), jnp.float32),
    )(summed, cnt2, x_tgt, wlT, bl2, wrT, a2)


def _layer(x_src, x_tgt, edge_index, ndst, Wl, bl, Wr, a):
    src = edge_index[0]
    dst = edge_index[1]
    if ndst == N3:
        seg, half, half_out = _SEG3S
        flat, cntf = seg(src, dst, x_src)
        return _tc_layer(flat, cntf[:, None], x_tgt, Wl.T, bl[None, :],
                         Wr.T, a[None, :], ndst)
    seg, quarter, quarter_out, nq = _SEGS[ndst]
    e_pad = _EPAD[ndst]
    padn = e_pad - src.shape[0]
    if padn:
        src = jnp.pad(src, (0, padn))
        dst = jnp.pad(dst, (0, padn), constant_values=-1)
    flat, cnt2 = seg(src, dst, x_src)
    if nq == 1 and quarter == quarter_out:
        summed = flat
    else:
        summed = jnp.concatenate(
            [flat[q * quarter_out:q * quarter_out + quarter]
             for q in range(nq)], axis=0)
    cntf = cnt2.reshape(-1)
    cl = CNT_ROWS * D
    cnt = jnp.concatenate(
        [cntf[q * cl:q * cl + quarter] for q in range(nq)])
    return _tc_layer(summed, cnt[:, None], x_tgt, Wl.T, bl[None, :], Wr.T,
                     a[None, :], ndst)


def kernel(x, edge_index1, edge_index2, edge_index3,
           Wl1, bl1, Wr1, a1, Wl2, bl2, Wr2, a2, Wl3, bl3, Wr3, a3):
    h1 = _layer(x, x[:N1], edge_index1, N1, Wl1, bl1, Wr1, a1)
    h2 = _layer(h1, h1[:N2], edge_index2, N2, Wl2, bl2, Wr2, a2)
    h3 = _layer(h2, h2[:N3], edge_index3, N3, Wl3, bl3, Wr3, a3)
    return h3
